# baseline (speedup 1.0000x reference)
import jax
import jax.numpy as jnp
from jax.experimental import pallas as pl

N = 10000
E = 160000
D = 128
NB = 10
RN = 16
NG = 64
L = 4
MAX_RADIUS = 5.0
INV_SQRT_NEIGH = 0.25


def _radial_basis(r):
    centers = jnp.linspace(0.0, MAX_RADIUS, NB)
    width = NB / MAX_RADIUS
    basis = jnp.exp(-((r[:, None] - centers[None, :]) * width) ** 2)
    cutoff = 0.5 * (jnp.cos(jnp.pi * jnp.clip(r / MAX_RADIUS, 0.0, 1.0)) + 1.0)
    return basis * cutoff[:, None]


def _gate_network(pos, x, edge_index, net, W1, W2, R1, b1, R2, b2, R3, b3):
    src = edge_index[0]
    dst = edge_index[1]
    edge_vec = pos[dst] - pos[src]
    r = jnp.sqrt(jnp.sum(edge_vec ** 2, axis=1) + 1e-12)
    basis = _radial_basis(r)
    h = x
    for l in range(L):
        hx = h @ W1[net, l]
        rad = jax.nn.silu(basis @ R1[net, l] + b1[net, l])
        rad = jax.nn.silu(rad @ R2[net, l] + b2[net, l])
        rad = rad @ R3[net, l] + b3[net, l]
        msg = hx[src] * rad
        agg = jax.ops.segment_sum(msg, dst, num_segments=N) * INV_SQRT_NEIGH
        h = agg @ W2[net, l]
        if l < L - 1:
            h = jax.nn.silu(h)
    return h


def _norm_apply_kernel(h_ref, norm_ref, out_ref):
    out_ref[...] = h_ref[...] / norm_ref[...]


def _normalize_sumsq(x, batch):
    ss = jnp.sum(x ** 2, axis=1)
    ssg = jax.ops.segment_sum(ss, batch, num_segments=NG)
    norm = jnp.sqrt(ssg + 1e-12)
    norm_per_node = norm[batch][:, None]
    return pl.pallas_call(
        _norm_apply_kernel,
        out_shape=jax.ShapeDtypeStruct((N, D), jnp.float32),
    )(x, norm_per_node)


def kernel(pos, x, batch, edge_index, pos_final_state, x_final_state,
           edge_index_final_state, pos_interpolated_transition_state,
           edge_index_interpolated_transition_state, p, W1, W2, R1, b1, R2,
           b2, R3, b3):
    out_i = _gate_network(pos, x, edge_index, 0, W1, W2, R1, b1, R2, b2, R3, b3)
    out_i = _normalize_sumsq(out_i, batch)
    out_f = _gate_network(pos_final_state, x_final_state,
                          edge_index_final_state, 1, W1, W2, R1, b1, R2, b2,
                          R3, b3)
    out_f = _normalize_sumsq(out_f, batch)
    pp = p[0]
    x_ts = (1.0 - pp) * out_i + pp * out_f
    out_ts = _gate_network(pos_interpolated_transition_state, x_ts,
                           edge_index_interpolated_transition_state, 2, W1,
                           W2, R1, b1, R2, b2, R3, b3)
    out_ts = _normalize_sumsq(out_ts, batch)
    return out_ts


# speedup vs baseline: 1.5864x; 1.5864x over previous
import functools

import jax
import jax.numpy as jnp
from jax import lax
from jax.experimental import pallas as pl
from jax.experimental.pallas import tpu as pltpu
from jax.experimental.pallas import tpu_sc as plsc

N = 10000
E = 160000
D = 128
NB = 10
RN = 16
NG = 64
L = 4
MAX_RADIUS = 5.0
INV_SQRT_NEIGH = 0.25

NC = 2
NS = 16
NW = NC * NS
EPT = 5120
E_PAD = EPT * NW
CH = 128
NCHUNK = EPT // CH
AGG_ROWS = 10240
ZROWS = AGG_ROWS // NS

def _sc_mesh():
    return plsc.VectorSubcoreMesh(core_axis_name="c", subcore_axis_name="s",
                                  num_cores=NC, num_subcores=NS)


PW = 128


def _sc_dvec_body(posp, srcb, dstb, dvec_out, psv, pdv, sv, dv, sem1, sem2):
    cid = lax.axis_index("c")
    sid = lax.axis_index("s")
    wid = cid * NS + sid
    base = wid * EPT
    pltpu.sync_copy(srcb.at[pl.ds(wid * NCHUNK, NCHUNK)], sv)
    pltpu.sync_copy(dstb.at[pl.ds(wid * NCHUNK, NCHUNK)], dv)

    def chunk(j, carry):
        c1 = pltpu.async_copy(posp.at[sv.at[j]], psv, sem1)
        c2 = pltpu.async_copy(posp.at[dv.at[j]], pdv, sem2)
        c1.wait()
        c2.wait()

        def diff(i, c2_):
            sl = pl.ds(0, 16)
            pdv[i, sl] = pdv[i, sl] - psv[i, sl]
            return c2_

        lax.fori_loop(0, CH, diff, 0)
        pltpu.sync_copy(pdv, dvec_out.at[pl.ds(base + j * CH, CH)])
        return carry

    lax.fori_loop(0, NCHUNK, chunk, 0)


def _sc_edge_dvec(*args):
    return pl.kernel(
        _sc_dvec_body,
        out_type=jax.ShapeDtypeStruct((E_PAD, PW), jnp.float32),
        mesh=_sc_mesh(),
        scratch_types=[
            pltpu.VMEM((CH, PW), jnp.float32),
            pltpu.VMEM((CH, PW), jnp.float32),
            pltpu.VMEM((NCHUNK, CH), jnp.int32),
            pltpu.VMEM((NCHUNK, CH), jnp.int32),
            pltpu.SemaphoreType.DMA,
            pltpu.SemaphoreType.DMA,
        ],
    )(*args)


def _sc_gms_body(hx, rad, srcb, dstb, agg2, hxv, radv, sv, dv, aggS, gsem):
    cid = lax.axis_index("c")
    sid = lax.axis_index("s")
    wid = cid * NS + sid
    base = wid * EPT

    def zbody(i, carry):
        for k in range(D // 16):
            hxv[i, pl.ds(k * 16, 16)] = jnp.zeros((16,), jnp.float32)
        return carry

    lax.fori_loop(0, CH, zbody, 0)
    for b in range(ZROWS // CH):
        pltpu.sync_copy(hxv, aggS.at[pl.ds(sid * ZROWS + b * CH, CH)])

    pltpu.sync_copy(srcb.at[pl.ds(wid * NCHUNK, NCHUNK)], sv)
    pltpu.sync_copy(dstb.at[pl.ds(wid * NCHUNK, NCHUNK)], dv)
    plsc.subcore_barrier()

    def chunk(j, carry):
        pltpu.async_copy(hx.at[sv.at[j]], hxv, gsem).wait()
        pltpu.sync_copy(rad.at[pl.ds(base + j * CH, CH)], radv)

        def mul(i, c2):
            for k in range(D // 16):
                sl = pl.ds(k * 16, 16)
                hxv[i, sl] = hxv[i, sl] * radv[i, sl]
            return c2

        lax.fori_loop(0, CH, mul, 0)
        pltpu.sync_copy(hxv, aggS.at[dv.at[j]], add=True)
        return carry

    lax.fori_loop(0, NCHUNK, chunk, 0)
    plsc.subcore_barrier()
    pltpu.sync_copy(aggS.at[pl.ds(sid * ZROWS, ZROWS)],
                    agg2.at[cid, pl.ds(sid * ZROWS, ZROWS)])


def _sc_gather_mul_scatter(*args):
    return pl.kernel(
        _sc_gms_body,
        out_type=jax.ShapeDtypeStruct((NC, AGG_ROWS, D), jnp.float32),
        mesh=_sc_mesh(),
        scratch_types=[
            pltpu.VMEM((CH, D), jnp.float32),
            pltpu.VMEM((CH, D), jnp.float32),
            pltpu.VMEM((NCHUNK, CH), jnp.int32),
            pltpu.VMEM((NCHUNK, CH), jnp.int32),
            pltpu.VMEM_SHARED((AGG_ROWS, D), jnp.float32),
            pltpu.SemaphoreType.DMA,
        ],
    )(*args)


_BE = 2048
_BN = 1000


def _rad_body(dv_ref, cen_ref, msk_ref, r1_ref, b1_ref, r2w_ref, b2_ref,
              r3_ref, b3_ref, out_ref):
    dv = dv_ref[...]
    r2 = jnp.sum(dv * dv, axis=1, keepdims=True)
    r = jnp.sqrt(r2 + 1e-12)
    width = NB / MAX_RADIUS
    basis = jnp.exp(-((r - cen_ref[...]) * width) ** 2)
    t = jnp.clip(r / MAX_RADIUS, 0.0, 1.0)
    cutoff = 0.5 * (jnp.cos(jnp.pi * t) + 1.0)
    basis = basis * cutoff * msk_ref[...]
    t1 = jax.nn.silu(
        jnp.dot(basis, r1_ref[...], preferred_element_type=jnp.float32)
        + b1_ref[...])
    t2 = jax.nn.silu(
        jnp.dot(t1, r2w_ref[...], preferred_element_type=jnp.float32)
        + b2_ref[...])
    out_ref[...] = (
        jnp.dot(t2, r3_ref[...], preferred_element_type=jnp.float32)
        + b3_ref[...])


def _tc_rad(dvec, centers_row, colmask_row, R1w, b1, R2w, b2, R3w, b3):
    grid = E_PAD // _BE
    return pl.pallas_call(
        _rad_body,
        grid=(grid,),
        in_specs=[
            pl.BlockSpec((_BE, PW), lambda g: (g, 0)),
            pl.BlockSpec((1, D), lambda g: (0, 0)),
            pl.BlockSpec((1, D), lambda g: (0, 0)),
            pl.BlockSpec((D, RN), lambda g: (0, 0)),
            pl.BlockSpec((1, RN), lambda g: (0, 0)),
            pl.BlockSpec((RN, RN), lambda g: (0, 0)),
            pl.BlockSpec((1, RN), lambda g: (0, 0)),
            pl.BlockSpec((RN, D), lambda g: (0, 0)),
            pl.BlockSpec((1, D), lambda g: (0, 0)),
        ],
        out_specs=pl.BlockSpec((_BE, D), lambda g: (g, 0)),
        out_shape=jax.ShapeDtypeStruct((E_PAD, D), jnp.float32),
    )(dvec, centers_row, colmask_row, R1w, b1, R2w, b2, R3w, b3)


def _hx0_body(x_ref, w_ref, out_ref):
    out_ref[...] = jnp.dot(x_ref[...], w_ref[...],
                           preferred_element_type=jnp.float32)


def _tc_hx0(x, W1l):
    return pl.pallas_call(
        _hx0_body,
        grid=(N // _BN,),
        in_specs=[
            pl.BlockSpec((_BN, D), lambda g: (g, 0)),
            pl.BlockSpec((D, D), lambda g: (0, 0)),
        ],
        out_specs=pl.BlockSpec((_BN, D), lambda g: (g, 0)),
        out_shape=jax.ShapeDtypeStruct((N, D), jnp.float32),
    )(x, W1l)


def _interp_hx0_body(oi_ref, of_ref, p_ref, w_ref, out_ref):
    pp = p_ref[0, 0]
    xts = (1.0 - pp) * oi_ref[...] + pp * of_ref[...]
    out_ref[...] = jnp.dot(xts, w_ref[...],
                           preferred_element_type=jnp.float32)


def _tc_interp_hx0(out_i, out_f, p2d, W1l):
    return pl.pallas_call(
        _interp_hx0_body,
        grid=(N // _BN,),
        in_specs=[
            pl.BlockSpec((_BN, D), lambda g: (g, 0)),
            pl.BlockSpec((_BN, D), lambda g: (g, 0)),
            pl.BlockSpec((1, 1), lambda g: (0, 0), memory_space=pltpu.SMEM),
            pl.BlockSpec((D, D), lambda g: (0, 0)),
        ],
        out_specs=pl.BlockSpec((_BN, D), lambda g: (g, 0)),
        out_shape=jax.ShapeDtypeStruct((N, D), jnp.float32),
    )(out_i, out_f, p2d, W1l)


def _mid_body(a0_ref, a1_ref, w2_ref, w1_ref, out_ref):
    agg = (a0_ref[0] + a1_ref[0]) * INV_SQRT_NEIGH
    h = jax.nn.silu(jnp.dot(agg, w2_ref[...],
                            preferred_element_type=jnp.float32))
    out_ref[...] = jnp.dot(h, w1_ref[...],
                           preferred_element_type=jnp.float32)


def _tc_mid(agg2, W2l, W1n):
    return pl.pallas_call(
        _mid_body,
        grid=(N // _BN,),
        in_specs=[
            pl.BlockSpec((1, _BN, D), lambda g: (0, g, 0)),
            pl.BlockSpec((1, _BN, D), lambda g: (1, g, 0)),
            pl.BlockSpec((D, D), lambda g: (0, 0)),
            pl.BlockSpec((D, D), lambda g: (0, 0)),
        ],
        out_specs=pl.BlockSpec((_BN, D), lambda g: (g, 0)),
        out_shape=jax.ShapeDtypeStruct((N, D), jnp.float32),
    )(agg2, agg2, W2l, W1n)


def _last_body(a0_ref, a1_ref, w2_ref, out_ref):
    agg = (a0_ref[0] + a1_ref[0]) * INV_SQRT_NEIGH
    out_ref[...] = jnp.dot(agg, w2_ref[...],
                           preferred_element_type=jnp.float32)


def _tc_last(agg2, W2l):
    return pl.pallas_call(
        _last_body,
        grid=(N // _BN,),
        in_specs=[
            pl.BlockSpec((1, _BN, D), lambda g: (0, g, 0)),
            pl.BlockSpec((1, _BN, D), lambda g: (1, g, 0)),
            pl.BlockSpec((D, D), lambda g: (0, 0)),
        ],
        out_specs=pl.BlockSpec((_BN, D), lambda g: (g, 0)),
        out_shape=jax.ShapeDtypeStruct((N, D), jnp.float32),
    )(agg2, agg2, W2l)


def _norm_body(x_ref, batch_ref, out_ref):
    x = x_ref[...]
    ss = jnp.sum(x * x, axis=1, keepdims=True)
    gid = lax.broadcasted_iota(jnp.int32, (N, NG), 1)
    seg = (batch_ref[...] == gid).astype(jnp.float32)
    ssg = jnp.sum(seg * ss, axis=0, keepdims=True)
    inv = 1.0 / jnp.sqrt(ssg + 1e-12)
    node_inv = jnp.sum(seg * inv, axis=1, keepdims=True)
    out_ref[...] = x * node_inv


def _tc_norm(x, batch2d):
    return pl.pallas_call(
        _norm_body,
        out_shape=jax.ShapeDtypeStruct((N, D), jnp.float32),
    )(x, batch2d)


def _prep_edges(edge_index):
    npad = E_PAD - E
    src = jnp.concatenate([edge_index[0], jnp.zeros((npad,), jnp.int32)])
    dst = jnp.concatenate([edge_index[1],
                           jnp.full((npad,), N, jnp.int32)])
    return src.reshape(E_PAD // CH, CH), dst.reshape(E_PAD // CH, CH)


def _prep_pos(pos):
    return jnp.pad(pos, ((0, AGG_ROWS - N), (0, PW - 3)))


def _gate_net(pos, x0_hx, edge_index, net, W1, W2, R1w, b1, R2, b2, R3, b3,
              centers_row, colmask_row):
    srcb, dstb = _prep_edges(edge_index)
    posp = _prep_pos(pos)
    dvec = _sc_edge_dvec(posp, srcb, dstb)
    hx = x0_hx
    for l in range(L):
        rad = _tc_rad(dvec, centers_row, colmask_row, R1w[net, l],
                      b1[net, l].reshape(1, RN), R2[net, l],
                      b2[net, l].reshape(1, RN), R3[net, l],
                      b3[net, l].reshape(1, D))
        agg2 = _sc_gather_mul_scatter(hx, rad, srcb, dstb)
        if l < L - 1:
            hx = _tc_mid(agg2, W2[net, l], W1[net, l + 1])
        else:
            h = _tc_last(agg2, W2[net, l])
    return h


def kernel(pos, x, batch, edge_index, pos_final_state, x_final_state,
           edge_index_final_state, pos_interpolated_transition_state,
           edge_index_interpolated_transition_state, p, W1, W2, R1, b1, R2,
           b2, R3, b3):
    centers = jnp.linspace(0.0, MAX_RADIUS, NB)
    centers_row = jnp.pad(centers, (0, D - NB)).reshape(1, D)
    colmask_row = (jnp.arange(D) < NB).astype(jnp.float32).reshape(1, D)
    R1w = jnp.pad(R1, ((0, 0), (0, 0), (0, D - NB), (0, 0)))
    batch2d = batch.astype(jnp.int32).reshape(N, 1)
    p2d = p.reshape(1, 1)

    hx0_i = _tc_hx0(x, W1[0, 0])
    h_i = _gate_net(pos, hx0_i, edge_index, 0, W1, W2, R1w, b1, R2, b2, R3,
                    b3, centers_row, colmask_row)
    out_i = _tc_norm(h_i, batch2d)

    hx0_f = _tc_hx0(x_final_state, W1[1, 0])
    h_f = _gate_net(pos_final_state, hx0_f, edge_index_final_state, 1, W1,
                    W2, R1w, b1, R2, b2, R3, b3, centers_row, colmask_row)
    out_f = _tc_norm(h_f, batch2d)

    hx0_t = _tc_interp_hx0(out_i, out_f, p2d, W1[2, 0])
    h_t = _gate_net(pos_interpolated_transition_state, hx0_t,
                    edge_index_interpolated_transition_state, 2, W1, W2,
                    R1w, b1, R2, b2, R3, b3, centers_row, colmask_row)
    out_ts = _tc_norm(h_t, batch2d)
    return out_ts


# speedup vs baseline: 1.6948x; 1.0684x over previous
import functools

import jax
import jax.numpy as jnp
from jax import lax
from jax.experimental import pallas as pl
from jax.experimental.pallas import tpu as pltpu
from jax.experimental.pallas import tpu_sc as plsc

N = 10000
E = 160000
D = 128
NB = 10
RN = 16
NG = 64
L = 4
MAX_RADIUS = 5.0
INV_SQRT_NEIGH = 0.25

NC = 2
NS = 16
NW = NC * NS
EPT = 5120
E_PAD = EPT * NW
CH = 64
NCHUNK = EPT // CH
AGG_ROWS = 10240
ZROWS = AGG_ROWS // NS

def _sc_mesh():
    return plsc.VectorSubcoreMesh(core_axis_name="c", subcore_axis_name="s",
                                  num_cores=NC, num_subcores=NS)


PW = 128
DW = 16


def _sc_dvec_body(posp, srcb, dstb, dvec_out,
                  ps0, pd0, ps1, pd1, d16, sv, dv, ss0, ds0, ss1, ds1):
    cid = lax.axis_index("c")
    sid = lax.axis_index("s")
    wid = cid * NS + sid
    base = wid * EPT
    pltpu.sync_copy(srcb.at[pl.ds(wid * NCHUNK, NCHUNK)], sv)
    pltpu.sync_copy(dstb.at[pl.ds(wid * NCHUNK, NCHUNK)], dv)

    bufs = ((ps0, pd0, ss0, ds0), (ps1, pd1, ss1, ds1))

    def issue(jj, b):
        ps, pd, ss, ds = bufs[b]
        pltpu.async_copy(posp.at[sv.at[jj]], ps, ss)
        pltpu.async_copy(posp.at[dv.at[jj]], pd, ds)

    def work(jj, b):
        ps, pd, ss, ds = bufs[b]
        pltpu.make_async_copy(posp.at[sv.at[jj]], ps, ss).wait()
        pltpu.make_async_copy(posp.at[dv.at[jj]], pd, ds).wait()

        @plsc.parallel_loop(0, CH, step=1, unroll=8)
        def diff(i):
            d16[i, :] = pd[i, pl.ds(0, DW)] - ps[i, pl.ds(0, DW)]

        pltpu.sync_copy(d16, dvec_out.at[pl.ds(base + jj * CH, CH)])

    issue(0, 0)
    issue(1, 1)

    def pair(j2, carry):
        for b in range(2):
            jj = j2 * 2 + b
            work(jj, b)
            issue(jj + 2, b)
        return carry

    lax.fori_loop(0, NCHUNK // 2 - 1, pair, 0)
    work(NCHUNK - 2, 0)
    work(NCHUNK - 1, 1)


def _sc_edge_dvec(*args):
    return pl.kernel(
        _sc_dvec_body,
        out_type=jax.ShapeDtypeStruct((E_PAD, DW), jnp.float32),
        mesh=_sc_mesh(),
        scratch_types=[
            pltpu.VMEM((CH, PW), jnp.float32),
            pltpu.VMEM((CH, PW), jnp.float32),
            pltpu.VMEM((CH, PW), jnp.float32),
            pltpu.VMEM((CH, PW), jnp.float32),
            pltpu.VMEM((CH, DW), jnp.float32),
            pltpu.VMEM((NCHUNK, CH), jnp.int32),
            pltpu.VMEM((NCHUNK, CH), jnp.int32),
            pltpu.SemaphoreType.DMA,
            pltpu.SemaphoreType.DMA,
            pltpu.SemaphoreType.DMA,
            pltpu.SemaphoreType.DMA,
        ],
    )(*args)


def _sc_gms_body(hx, rad, srcb, dstb, agg2,
                 hx0, rd0, hx1, sv, dv, aggS, gs0, gs1):
    cid = lax.axis_index("c")
    sid = lax.axis_index("s")
    wid = cid * NS + sid
    base = wid * EPT

    def zbody(i, carry):
        for k in range(D // 16):
            hx0[i, pl.ds(k * 16, 16)] = jnp.zeros((16,), jnp.float32)
        return carry

    lax.fori_loop(0, CH, zbody, 0)
    for b in range(ZROWS // CH):
        pltpu.sync_copy(hx0, aggS.at[pl.ds(sid * ZROWS + b * CH, CH)])

    pltpu.sync_copy(srcb.at[pl.ds(wid * NCHUNK, NCHUNK)], sv)
    pltpu.sync_copy(dstb.at[pl.ds(wid * NCHUNK, NCHUNK)], dv)
    plsc.subcore_barrier()

    bufs = ((hx0, gs0), (hx1, gs1))

    def issue(jj, b):
        hxv, gs = bufs[b]
        pltpu.async_copy(hx.at[sv.at[jj]], hxv, gs)

    def work(jj, b):
        hxv, gs = bufs[b]
        pltpu.sync_copy(rad.at[pl.ds(base + jj * CH, CH)], rd0)
        pltpu.make_async_copy(hx.at[sv.at[jj]], hxv, gs).wait()

        @plsc.parallel_loop(0, CH, step=1, unroll=2)
        def mul(i):
            for k in range(D // 16):
                sl = pl.ds(k * 16, 16)
                hxv[i, sl] = hxv[i, sl] * rd0[i, sl]

        pltpu.sync_copy(hxv, aggS.at[dv.at[jj]], add=True)

    issue(0, 0)
    issue(1, 1)

    def pair(j2, carry):
        for b in range(2):
            jj = j2 * 2 + b
            work(jj, b)
            issue(jj + 2, b)
        return carry

    lax.fori_loop(0, NCHUNK // 2 - 1, pair, 0)
    work(NCHUNK - 2, 0)
    work(NCHUNK - 1, 1)

    plsc.subcore_barrier()
    pltpu.sync_copy(aggS.at[pl.ds(sid * ZROWS, ZROWS)],
                    agg2.at[cid, pl.ds(sid * ZROWS, ZROWS)])


def _sc_gather_mul_scatter(*args):
    return pl.kernel(
        _sc_gms_body,
        out_type=jax.ShapeDtypeStruct((NC, AGG_ROWS, D), jnp.float32),
        mesh=_sc_mesh(),
        scratch_types=[
            pltpu.VMEM((CH, D), jnp.float32),
            pltpu.VMEM((CH, D), jnp.float32),
            pltpu.VMEM((CH, D), jnp.float32),
            pltpu.VMEM((NCHUNK, CH), jnp.int32),
            pltpu.VMEM((NCHUNK, CH), jnp.int32),
            pltpu.VMEM_SHARED((AGG_ROWS, D), jnp.float32),
            pltpu.SemaphoreType.DMA,
            pltpu.SemaphoreType.DMA,
        ],
    )(*args)


_BE = 2048
_BN = 1000


def _rad_body(dv_ref, cen_ref, msk_ref, r1_ref, b1_ref, r2w_ref, b2_ref,
              r3_ref, b3_ref, out_ref):
    dv = dv_ref[...]
    r2 = jnp.sum(dv * dv, axis=1, keepdims=True)
    r = jnp.sqrt(r2 + 1e-12)
    width = NB / MAX_RADIUS
    basis = jnp.exp(-((r - cen_ref[...]) * width) ** 2)
    t = jnp.clip(r / MAX_RADIUS, 0.0, 1.0)
    cutoff = 0.5 * (jnp.cos(jnp.pi * t) + 1.0)
    basis = basis * cutoff * msk_ref[...]
    t1 = jax.nn.silu(
        jnp.dot(basis, r1_ref[...], preferred_element_type=jnp.float32)
        + b1_ref[...])
    t2 = jax.nn.silu(
        jnp.dot(t1, r2w_ref[...], preferred_element_type=jnp.float32)
        + b2_ref[...])
    out_ref[...] = (
        jnp.dot(t2, r3_ref[...], preferred_element_type=jnp.float32)
        + b3_ref[...])


def _tc_rad(dvec, centers_row, colmask_row, R1w, b1, R2w, b2, R3w, b3):
    grid = E_PAD // _BE
    return pl.pallas_call(
        _rad_body,
        grid=(grid,),
        in_specs=[
            pl.BlockSpec((_BE, DW), lambda g: (g, 0)),
            pl.BlockSpec((1, D), lambda g: (0, 0)),
            pl.BlockSpec((1, D), lambda g: (0, 0)),
            pl.BlockSpec((D, RN), lambda g: (0, 0)),
            pl.BlockSpec((1, RN), lambda g: (0, 0)),
            pl.BlockSpec((RN, RN), lambda g: (0, 0)),
            pl.BlockSpec((1, RN), lambda g: (0, 0)),
            pl.BlockSpec((RN, D), lambda g: (0, 0)),
            pl.BlockSpec((1, D), lambda g: (0, 0)),
        ],
        out_specs=pl.BlockSpec((_BE, D), lambda g: (g, 0)),
        out_shape=jax.ShapeDtypeStruct((E_PAD, D), jnp.float32),
    )(dvec, centers_row, colmask_row, R1w, b1, R2w, b2, R3w, b3)


def _hx0_body(x_ref, w_ref, out_ref):
    out_ref[...] = jnp.dot(x_ref[...], w_ref[...],
                           preferred_element_type=jnp.float32)


def _tc_hx0(x, W1l):
    return pl.pallas_call(
        _hx0_body,
        grid=(N // _BN,),
        in_specs=[
            pl.BlockSpec((_BN, D), lambda g: (g, 0)),
            pl.BlockSpec((D, D), lambda g: (0, 0)),
        ],
        out_specs=pl.BlockSpec((_BN, D), lambda g: (g, 0)),
        out_shape=jax.ShapeDtypeStruct((N, D), jnp.float32),
    )(x, W1l)


def _interp_hx0_body(oi_ref, of_ref, p_ref, w_ref, out_ref):
    pp = p_ref[0, 0]
    xts = (1.0 - pp) * oi_ref[...] + pp * of_ref[...]
    out_ref[...] = jnp.dot(xts, w_ref[...],
                           preferred_element_type=jnp.float32)


def _tc_interp_hx0(out_i, out_f, p2d, W1l):
    return pl.pallas_call(
        _interp_hx0_body,
        grid=(N // _BN,),
        in_specs=[
            pl.BlockSpec((_BN, D), lambda g: (g, 0)),
            pl.BlockSpec((_BN, D), lambda g: (g, 0)),
            pl.BlockSpec((1, 1), lambda g: (0, 0), memory_space=pltpu.SMEM),
            pl.BlockSpec((D, D), lambda g: (0, 0)),
        ],
        out_specs=pl.BlockSpec((_BN, D), lambda g: (g, 0)),
        out_shape=jax.ShapeDtypeStruct((N, D), jnp.float32),
    )(out_i, out_f, p2d, W1l)


def _mid_body(a0_ref, a1_ref, w2_ref, w1_ref, out_ref):
    agg = (a0_ref[0] + a1_ref[0]) * INV_SQRT_NEIGH
    h = jax.nn.silu(jnp.dot(agg, w2_ref[...],
                            preferred_element_type=jnp.float32))
    out_ref[...] = jnp.dot(h, w1_ref[...],
                           preferred_element_type=jnp.float32)


def _tc_mid(agg2, W2l, W1n):
    return pl.pallas_call(
        _mid_body,
        grid=(N // _BN,),
        in_specs=[
            pl.BlockSpec((1, _BN, D), lambda g: (0, g, 0)),
            pl.BlockSpec((1, _BN, D), lambda g: (1, g, 0)),
            pl.BlockSpec((D, D), lambda g: (0, 0)),
            pl.BlockSpec((D, D), lambda g: (0, 0)),
        ],
        out_specs=pl.BlockSpec((_BN, D), lambda g: (g, 0)),
        out_shape=jax.ShapeDtypeStruct((N, D), jnp.float32),
    )(agg2, agg2, W2l, W1n)


def _last_body(a0_ref, a1_ref, w2_ref, out_ref):
    agg = (a0_ref[0] + a1_ref[0]) * INV_SQRT_NEIGH
    out_ref[...] = jnp.dot(agg, w2_ref[...],
                           preferred_element_type=jnp.float32)


def _tc_last(agg2, W2l):
    return pl.pallas_call(
        _last_body,
        grid=(N // _BN,),
        in_specs=[
            pl.BlockSpec((1, _BN, D), lambda g: (0, g, 0)),
            pl.BlockSpec((1, _BN, D), lambda g: (1, g, 0)),
            pl.BlockSpec((D, D), lambda g: (0, 0)),
        ],
        out_specs=pl.BlockSpec((_BN, D), lambda g: (g, 0)),
        out_shape=jax.ShapeDtypeStruct((N, D), jnp.float32),
    )(agg2, agg2, W2l)


def _norm_body(x_ref, batch_ref, out_ref):
    x = x_ref[...]
    ss = jnp.sum(x * x, axis=1, keepdims=True)
    gid = lax.broadcasted_iota(jnp.int32, (N, NG), 1)
    seg = (batch_ref[...] == gid).astype(jnp.float32)
    ssg = jnp.sum(seg * ss, axis=0, keepdims=True)
    inv = 1.0 / jnp.sqrt(ssg + 1e-12)
    node_inv = jnp.sum(seg * inv, axis=1, keepdims=True)
    out_ref[...] = x * node_inv


def _tc_norm(x, batch2d):
    return pl.pallas_call(
        _norm_body,
        out_shape=jax.ShapeDtypeStruct((N, D), jnp.float32),
    )(x, batch2d)


def _prep_edges(edge_index):
    npad = E_PAD - E
    src = jnp.concatenate([edge_index[0], jnp.zeros((npad,), jnp.int32)])
    dst = jnp.concatenate([edge_index[1],
                           jnp.full((npad,), N, jnp.int32)])
    return src.reshape(E_PAD // CH, CH), dst.reshape(E_PAD // CH, CH)


def _prep_pos(pos):
    return jnp.pad(pos, ((0, AGG_ROWS - N), (0, PW - 3)))


def _gate_net(pos, x0_hx, edge_index, net, W1, W2, R1w, b1, R2, b2, R3, b3,
              centers_row, colmask_row):
    srcb, dstb = _prep_edges(edge_index)
    posp = _prep_pos(pos)
    dvec = _sc_edge_dvec(posp, srcb, dstb)
    hx = x0_hx
    for l in range(L):
        rad = _tc_rad(dvec, centers_row, colmask_row, R1w[net, l],
                      b1[net, l].reshape(1, RN), R2[net, l],
                      b2[net, l].reshape(1, RN), R3[net, l],
                      b3[net, l].reshape(1, D))
        agg2 = _sc_gather_mul_scatter(hx, rad, srcb, dstb)
        if l < L - 1:
            hx = _tc_mid(agg2, W2[net, l], W1[net, l + 1])
        else:
            h = _tc_last(agg2, W2[net, l])
    return h


def kernel(pos, x, batch, edge_index, pos_final_state, x_final_state,
           edge_index_final_state, pos_interpolated_transition_state,
           edge_index_interpolated_transition_state, p, W1, W2, R1, b1, R2,
           b2, R3, b3):
    centers = jnp.linspace(0.0, MAX_RADIUS, NB)
    centers_row = jnp.pad(centers, (0, D - NB)).reshape(1, D)
    colmask_row = (jnp.arange(D) < NB).astype(jnp.float32).reshape(1, D)
    R1w = jnp.pad(R1, ((0, 0), (0, 0), (0, D - NB), (0, 0)))
    batch2d = batch.astype(jnp.int32).reshape(N, 1)
    p2d = p.reshape(1, 1)

    hx0_i = _tc_hx0(x, W1[0, 0])
    h_i = _gate_net(pos, hx0_i, edge_index, 0, W1, W2, R1w, b1, R2, b2, R3,
                    b3, centers_row, colmask_row)
    out_i = _tc_norm(h_i, batch2d)

    hx0_f = _tc_hx0(x_final_state, W1[1, 0])
    h_f = _gate_net(pos_final_state, hx0_f, edge_index_final_state, 1, W1,
                    W2, R1w, b1, R2, b2, R3, b3, centers_row, colmask_row)
    out_f = _tc_norm(h_f, batch2d)

    hx0_t = _tc_interp_hx0(out_i, out_f, p2d, W1[2, 0])
    h_t = _gate_net(pos_interpolated_transition_state, hx0_t,
                    edge_index_interpolated_transition_state, 2, W1, W2,
                    R1w, b1, R2, b2, R3, b3, centers_row, colmask_row)
    out_ts = _tc_norm(h_t, batch2d)
    return out_ts


# speedup vs baseline: 1.7019x; 1.0042x over previous
import functools

import jax
import jax.numpy as jnp
from jax import lax
from jax.experimental import pallas as pl
from jax.experimental.pallas import tpu as pltpu
from jax.experimental.pallas import tpu_sc as plsc

N = 10000
E = 160000
D = 128
NB = 10
RN = 16
NG = 64
L = 4
MAX_RADIUS = 5.0
INV_SQRT_NEIGH = 0.25

NC = 2
NS = 16
NW = NC * NS
EPT = 5120
E_PAD = EPT * NW
CH = 64
NCHUNK = EPT // CH
AGG_ROWS = 10240
ZROWS = AGG_ROWS // NS

def _sc_mesh():
    return plsc.VectorSubcoreMesh(core_axis_name="c", subcore_axis_name="s",
                                  num_cores=NC, num_subcores=NS)


PW = 128
CHD = 128
NCHUNKD = EPT // CHD
DW = 16


def _sc_dvec_body(posp, srcb, dstb, dvec_out,
                  ps0, pd0, ps1, pd1, d16, sv, dv, ss0, ds0, ss1, ds1):
    cid = lax.axis_index("c")
    sid = lax.axis_index("s")
    wid = cid * NS + sid
    base = wid * EPT
    pltpu.sync_copy(srcb.at[pl.ds(wid * NCHUNKD, NCHUNKD)], sv)
    pltpu.sync_copy(dstb.at[pl.ds(wid * NCHUNKD, NCHUNKD)], dv)

    bufs = ((ps0, pd0, ss0, ds0), (ps1, pd1, ss1, ds1))

    def issue(jj, b):
        ps, pd, ss, ds = bufs[b]
        pltpu.async_copy(posp.at[sv.at[jj]], ps, ss)
        pltpu.async_copy(posp.at[dv.at[jj]], pd, ds)

    def work(jj, b):
        ps, pd, ss, ds = bufs[b]
        pltpu.make_async_copy(posp.at[sv.at[jj]], ps, ss).wait()
        pltpu.make_async_copy(posp.at[dv.at[jj]], pd, ds).wait()

        @plsc.parallel_loop(0, CHD, step=1, unroll=8)
        def diff(i):
            d16[i, :] = pd[i, pl.ds(0, DW)] - ps[i, pl.ds(0, DW)]

        pltpu.sync_copy(d16, dvec_out.at[pl.ds(base + jj * CHD, CHD)])

    issue(0, 0)
    issue(1, 1)

    def pair(j2, carry):
        for b in range(2):
            jj = j2 * 2 + b
            work(jj, b)
            issue(jj + 2, b)
        return carry

    lax.fori_loop(0, NCHUNKD // 2 - 1, pair, 0)
    work(NCHUNKD - 2, 0)
    work(NCHUNKD - 1, 1)


def _sc_edge_dvec(*args):
    return pl.kernel(
        _sc_dvec_body,
        out_type=jax.ShapeDtypeStruct((E_PAD, DW), jnp.float32),
        mesh=_sc_mesh(),
        scratch_types=[
            pltpu.VMEM((CHD, PW), jnp.float32),
            pltpu.VMEM((CHD, PW), jnp.float32),
            pltpu.VMEM((CHD, PW), jnp.float32),
            pltpu.VMEM((CHD, PW), jnp.float32),
            pltpu.VMEM((CHD, DW), jnp.float32),
            pltpu.VMEM((NCHUNKD, CHD), jnp.int32),
            pltpu.VMEM((NCHUNKD, CHD), jnp.int32),
            pltpu.SemaphoreType.DMA,
            pltpu.SemaphoreType.DMA,
            pltpu.SemaphoreType.DMA,
            pltpu.SemaphoreType.DMA,
        ],
    )(*args)


def _sc_gms_body(hx, rad, srcb, dstb, agg2,
                 hx0, rd0, hx1, sv, dv, aggS, gs0, gs1):
    cid = lax.axis_index("c")
    sid = lax.axis_index("s")
    wid = cid * NS + sid
    base = wid * EPT

    def zbody(i, carry):
        for k in range(D // 16):
            hx0[i, pl.ds(k * 16, 16)] = jnp.zeros((16,), jnp.float32)
        return carry

    lax.fori_loop(0, CH, zbody, 0)
    for b in range(ZROWS // CH):
        pltpu.sync_copy(hx0, aggS.at[pl.ds(sid * ZROWS + b * CH, CH)])

    pltpu.sync_copy(srcb.at[pl.ds(wid * NCHUNK, NCHUNK)], sv)
    pltpu.sync_copy(dstb.at[pl.ds(wid * NCHUNK, NCHUNK)], dv)
    plsc.subcore_barrier()

    bufs = ((hx0, gs0), (hx1, gs1))

    def issue(jj, b):
        hxv, gs = bufs[b]
        pltpu.async_copy(hx.at[sv.at[jj]], hxv, gs)

    def work(jj, b):
        hxv, gs = bufs[b]
        pltpu.sync_copy(rad.at[pl.ds(base + jj * CH, CH)], rd0)
        pltpu.make_async_copy(hx.at[sv.at[jj]], hxv, gs).wait()

        @plsc.parallel_loop(0, CH, step=1, unroll=2)
        def mul(i):
            for k in range(D // 16):
                sl = pl.ds(k * 16, 16)
                hxv[i, sl] = hxv[i, sl] * rd0[i, sl]

        pltpu.sync_copy(hxv, aggS.at[dv.at[jj]], add=True)

    issue(0, 0)
    issue(1, 1)

    def pair(j2, carry):
        for b in range(2):
            jj = j2 * 2 + b
            work(jj, b)
            issue(jj + 2, b)
        return carry

    lax.fori_loop(0, NCHUNK // 2 - 1, pair, 0)
    work(NCHUNK - 2, 0)
    work(NCHUNK - 1, 1)

    plsc.subcore_barrier()
    pltpu.sync_copy(aggS.at[pl.ds(sid * ZROWS, ZROWS)],
                    agg2.at[cid, pl.ds(sid * ZROWS, ZROWS)])


def _sc_gather_mul_scatter(*args):
    return pl.kernel(
        _sc_gms_body,
        out_type=jax.ShapeDtypeStruct((NC, AGG_ROWS, D), jnp.float32),
        mesh=_sc_mesh(),
        scratch_types=[
            pltpu.VMEM((CH, D), jnp.float32),
            pltpu.VMEM((CH, D), jnp.float32),
            pltpu.VMEM((CH, D), jnp.float32),
            pltpu.VMEM((NCHUNK, CH), jnp.int32),
            pltpu.VMEM((NCHUNK, CH), jnp.int32),
            pltpu.VMEM_SHARED((AGG_ROWS, D), jnp.float32),
            pltpu.SemaphoreType.DMA,
            pltpu.SemaphoreType.DMA,
        ],
    )(*args)


_BE = 2048
_BN = 1000


def _rad_body(dv_ref, cen_ref, msk_ref, r1_ref, b1_ref, r2w_ref, b2_ref,
              r3_ref, b3_ref, out_ref):
    dv = dv_ref[...]
    r2 = jnp.sum(dv * dv, axis=1, keepdims=True)
    r = jnp.sqrt(r2 + 1e-12)
    width = NB / MAX_RADIUS
    basis = jnp.exp(-((r - cen_ref[...]) * width) ** 2)
    t = jnp.clip(r / MAX_RADIUS, 0.0, 1.0)
    cutoff = 0.5 * (jnp.cos(jnp.pi * t) + 1.0)
    basis = basis * cutoff * msk_ref[...]
    t1 = jax.nn.silu(
        jnp.dot(basis, r1_ref[...], preferred_element_type=jnp.float32)
        + b1_ref[...])
    t2 = jax.nn.silu(
        jnp.dot(t1, r2w_ref[...], preferred_element_type=jnp.float32)
        + b2_ref[...])
    out_ref[...] = (
        jnp.dot(t2, r3_ref[...], preferred_element_type=jnp.float32)
        + b3_ref[...])


def _tc_rad(dvec, centers_row, colmask_row, R1w, b1, R2w, b2, R3w, b3):
    grid = E_PAD // _BE
    return pl.pallas_call(
        _rad_body,
        grid=(grid,),
        in_specs=[
            pl.BlockSpec((_BE, DW), lambda g: (g, 0)),
            pl.BlockSpec((1, D), lambda g: (0, 0)),
            pl.BlockSpec((1, D), lambda g: (0, 0)),
            pl.BlockSpec((D, RN), lambda g: (0, 0)),
            pl.BlockSpec((1, RN), lambda g: (0, 0)),
            pl.BlockSpec((RN, RN), lambda g: (0, 0)),
            pl.BlockSpec((1, RN), lambda g: (0, 0)),
            pl.BlockSpec((RN, D), lambda g: (0, 0)),
            pl.BlockSpec((1, D), lambda g: (0, 0)),
        ],
        out_specs=pl.BlockSpec((_BE, D), lambda g: (g, 0)),
        out_shape=jax.ShapeDtypeStruct((E_PAD, D), jnp.float32),
    )(dvec, centers_row, colmask_row, R1w, b1, R2w, b2, R3w, b3)


def _hx0_body(x_ref, w_ref, out_ref):
    out_ref[...] = jnp.dot(x_ref[...], w_ref[...],
                           preferred_element_type=jnp.float32)


def _tc_hx0(x, W1l):
    return pl.pallas_call(
        _hx0_body,
        grid=(N // _BN,),
        in_specs=[
            pl.BlockSpec((_BN, D), lambda g: (g, 0)),
            pl.BlockSpec((D, D), lambda g: (0, 0)),
        ],
        out_specs=pl.BlockSpec((_BN, D), lambda g: (g, 0)),
        out_shape=jax.ShapeDtypeStruct((N, D), jnp.float32),
    )(x, W1l)


def _interp_hx0_body(oi_ref, of_ref, p_ref, w_ref, out_ref):
    pp = p_ref[0, 0]
    xts = (1.0 - pp) * oi_ref[...] + pp * of_ref[...]
    out_ref[...] = jnp.dot(xts, w_ref[...],
                           preferred_element_type=jnp.float32)


def _tc_interp_hx0(out_i, out_f, p2d, W1l):
    return pl.pallas_call(
        _interp_hx0_body,
        grid=(N // _BN,),
        in_specs=[
            pl.BlockSpec((_BN, D), lambda g: (g, 0)),
            pl.BlockSpec((_BN, D), lambda g: (g, 0)),
            pl.BlockSpec((1, 1), lambda g: (0, 0), memory_space=pltpu.SMEM),
            pl.BlockSpec((D, D), lambda g: (0, 0)),
        ],
        out_specs=pl.BlockSpec((_BN, D), lambda g: (g, 0)),
        out_shape=jax.ShapeDtypeStruct((N, D), jnp.float32),
    )(out_i, out_f, p2d, W1l)


def _mid_body(a0_ref, a1_ref, w2_ref, w1_ref, out_ref):
    agg = (a0_ref[0] + a1_ref[0]) * INV_SQRT_NEIGH
    h = jax.nn.silu(jnp.dot(agg, w2_ref[...],
                            preferred_element_type=jnp.float32))
    out_ref[...] = jnp.dot(h, w1_ref[...],
                           preferred_element_type=jnp.float32)


def _tc_mid(agg2, W2l, W1n):
    return pl.pallas_call(
        _mid_body,
        grid=(N // _BN,),
        in_specs=[
            pl.BlockSpec((1, _BN, D), lambda g: (0, g, 0)),
            pl.BlockSpec((1, _BN, D), lambda g: (1, g, 0)),
            pl.BlockSpec((D, D), lambda g: (0, 0)),
            pl.BlockSpec((D, D), lambda g: (0, 0)),
        ],
        out_specs=pl.BlockSpec((_BN, D), lambda g: (g, 0)),
        out_shape=jax.ShapeDtypeStruct((N, D), jnp.float32),
    )(agg2, agg2, W2l, W1n)


def _last_body(a0_ref, a1_ref, w2_ref, out_ref):
    agg = (a0_ref[0] + a1_ref[0]) * INV_SQRT_NEIGH
    out_ref[...] = jnp.dot(agg, w2_ref[...],
                           preferred_element_type=jnp.float32)


def _tc_last(agg2, W2l):
    return pl.pallas_call(
        _last_body,
        grid=(N // _BN,),
        in_specs=[
            pl.BlockSpec((1, _BN, D), lambda g: (0, g, 0)),
            pl.BlockSpec((1, _BN, D), lambda g: (1, g, 0)),
            pl.BlockSpec((D, D), lambda g: (0, 0)),
        ],
        out_specs=pl.BlockSpec((_BN, D), lambda g: (g, 0)),
        out_shape=jax.ShapeDtypeStruct((N, D), jnp.float32),
    )(agg2, agg2, W2l)


def _norm_body(x_ref, batch_ref, out_ref):
    x = x_ref[...]
    ss = jnp.sum(x * x, axis=1, keepdims=True)
    gid = lax.broadcasted_iota(jnp.int32, (N, NG), 1)
    seg = (batch_ref[...] == gid).astype(jnp.float32)
    ssg = jnp.sum(seg * ss, axis=0, keepdims=True)
    inv = 1.0 / jnp.sqrt(ssg + 1e-12)
    node_inv = jnp.sum(seg * inv, axis=1, keepdims=True)
    out_ref[...] = x * node_inv


def _tc_norm(x, batch2d):
    return pl.pallas_call(
        _norm_body,
        out_shape=jax.ShapeDtypeStruct((N, D), jnp.float32),
    )(x, batch2d)


def _prep_edges(edge_index):
    npad = E_PAD - E
    src = jnp.concatenate([edge_index[0], jnp.zeros((npad,), jnp.int32)])
    dst = jnp.concatenate([edge_index[1],
                           jnp.full((npad,), N, jnp.int32)])
    return src, dst


def _prep_pos(pos):
    return jnp.pad(pos, ((0, AGG_ROWS - N), (0, PW - 3)))


def _gate_net(pos, x0_hx, edge_index, net, W1, W2, R1w, b1, R2, b2, R3, b3,
              centers_row, colmask_row):
    src, dst = _prep_edges(edge_index)
    srcb = src.reshape(E_PAD // CH, CH)
    dstb = dst.reshape(E_PAD // CH, CH)
    posp = _prep_pos(pos)
    dvec = _sc_edge_dvec(posp, src.reshape(E_PAD // CHD, CHD),
                         dst.reshape(E_PAD // CHD, CHD))
    hx = x0_hx
    for l in range(L):
        rad = _tc_rad(dvec, centers_row, colmask_row, R1w[net, l],
                      b1[net, l].reshape(1, RN), R2[net, l],
                      b2[net, l].reshape(1, RN), R3[net, l],
                      b3[net, l].reshape(1, D))
        agg2 = _sc_gather_mul_scatter(hx, rad, srcb, dstb)
        if l < L - 1:
            hx = _tc_mid(agg2, W2[net, l], W1[net, l + 1])
        else:
            h = _tc_last(agg2, W2[net, l])
    return h


def kernel(pos, x, batch, edge_index, pos_final_state, x_final_state,
           edge_index_final_state, pos_interpolated_transition_state,
           edge_index_interpolated_transition_state, p, W1, W2, R1, b1, R2,
           b2, R3, b3):
    centers = jnp.linspace(0.0, MAX_RADIUS, NB)
    centers_row = jnp.pad(centers, (0, D - NB)).reshape(1, D)
    colmask_row = (jnp.arange(D) < NB).astype(jnp.float32).reshape(1, D)
    R1w = jnp.pad(R1, ((0, 0), (0, 0), (0, D - NB), (0, 0)))
    batch2d = batch.astype(jnp.int32).reshape(N, 1)
    p2d = p.reshape(1, 1)

    hx0_i = _tc_hx0(x, W1[0, 0])
    h_i = _gate_net(pos, hx0_i, edge_index, 0, W1, W2, R1w, b1, R2, b2, R3,
                    b3, centers_row, colmask_row)
    out_i = _tc_norm(h_i, batch2d)

    hx0_f = _tc_hx0(x_final_state, W1[1, 0])
    h_f = _gate_net(pos_final_state, hx0_f, edge_index_final_state, 1, W1,
                    W2, R1w, b1, R2, b2, R3, b3, centers_row, colmask_row)
    out_f = _tc_norm(h_f, batch2d)

    hx0_t = _tc_interp_hx0(out_i, out_f, p2d, W1[2, 0])
    h_t = _gate_net(pos_interpolated_transition_state, hx0_t,
                    edge_index_interpolated_transition_state, 2, W1, W2,
                    R1w, b1, R2, b2, R3, b3, centers_row, colmask_row)
    out_ts = _tc_norm(h_t, batch2d)
    return out_ts


# speedup vs baseline: 1.7035x; 1.0010x over previous
import functools

import jax
import jax.numpy as jnp
from jax import lax
from jax.experimental import pallas as pl
from jax.experimental.pallas import tpu as pltpu
from jax.experimental.pallas import tpu_sc as plsc

N = 10000
E = 160000
D = 128
NB = 10
RN = 16
NG = 64
L = 4
MAX_RADIUS = 5.0
INV_SQRT_NEIGH = 0.25

NC = 2
NS = 16
NW = NC * NS
EPT = 5120
E_PAD = EPT * NW
CH = 64
NCHUNK = EPT // CH
AGG_ROWS = 10240
ZROWS = AGG_ROWS // NS

def _sc_mesh():
    return plsc.VectorSubcoreMesh(core_axis_name="c", subcore_axis_name="s",
                                  num_cores=NC, num_subcores=NS)


PW = 128
CHD = 128
NCHUNKD = EPT // CHD
DW = 16


def _sc_dvec_body(posp, srcb, dstb, dvec_out,
                  ps0, pd0, ps1, pd1, d16, sv, dv, ss0, ds0, ss1, ds1):
    cid = lax.axis_index("c")
    sid = lax.axis_index("s")
    wid = cid * NS + sid
    base = wid * EPT
    pltpu.sync_copy(srcb.at[pl.ds(wid * NCHUNKD, NCHUNKD)], sv)
    pltpu.sync_copy(dstb.at[pl.ds(wid * NCHUNKD, NCHUNKD)], dv)

    bufs = ((ps0, pd0, ss0, ds0), (ps1, pd1, ss1, ds1))

    def issue(jj, b):
        ps, pd, ss, ds = bufs[b]
        pltpu.async_copy(posp.at[sv.at[jj]], ps, ss)
        pltpu.async_copy(posp.at[dv.at[jj]], pd, ds)

    def work(jj, b):
        ps, pd, ss, ds = bufs[b]
        pltpu.make_async_copy(posp.at[sv.at[jj]], ps, ss).wait()
        pltpu.make_async_copy(posp.at[dv.at[jj]], pd, ds).wait()

        @plsc.parallel_loop(0, CHD, step=1, unroll=8)
        def diff(i):
            d16[i, :] = pd[i, pl.ds(0, DW)] - ps[i, pl.ds(0, DW)]

        pltpu.sync_copy(d16, dvec_out.at[pl.ds(base + jj * CHD, CHD)])

    issue(0, 0)
    issue(1, 1)

    def pair(j2, carry):
        for b in range(2):
            jj = j2 * 2 + b
            work(jj, b)
            issue(jj + 2, b)
        return carry

    lax.fori_loop(0, NCHUNKD // 2 - 1, pair, 0)
    work(NCHUNKD - 2, 0)
    work(NCHUNKD - 1, 1)


def _sc_edge_dvec(*args):
    return pl.kernel(
        _sc_dvec_body,
        out_type=jax.ShapeDtypeStruct((E_PAD, DW), jnp.float32),
        mesh=_sc_mesh(),
        scratch_types=[
            pltpu.VMEM((CHD, PW), jnp.float32),
            pltpu.VMEM((CHD, PW), jnp.float32),
            pltpu.VMEM((CHD, PW), jnp.float32),
            pltpu.VMEM((CHD, PW), jnp.float32),
            pltpu.VMEM((CHD, DW), jnp.float32),
            pltpu.VMEM((NCHUNKD, CHD), jnp.int32),
            pltpu.VMEM((NCHUNKD, CHD), jnp.int32),
            pltpu.SemaphoreType.DMA,
            pltpu.SemaphoreType.DMA,
            pltpu.SemaphoreType.DMA,
            pltpu.SemaphoreType.DMA,
        ],
    )(*args)


def _sc_gms_body(hx, rad, srcb, dstb, agg2,
                 hx0, rd0, hx1, sv, dv, aggS, gs0, gs1):
    cid = lax.axis_index("c")
    sid = lax.axis_index("s")
    wid = cid * NS + sid
    base = wid * EPT

    def zbody(i, carry):
        for k in range(D // 16):
            hx0[i, pl.ds(k * 16, 16)] = jnp.zeros((16,), jnp.float32)
        return carry

    lax.fori_loop(0, CH, zbody, 0)
    for b in range(ZROWS // CH):
        pltpu.sync_copy(hx0, aggS.at[pl.ds(sid * ZROWS + b * CH, CH)])

    pltpu.sync_copy(srcb.at[pl.ds(wid * NCHUNK, NCHUNK)], sv)
    pltpu.sync_copy(dstb.at[pl.ds(wid * NCHUNK, NCHUNK)], dv)
    plsc.subcore_barrier()

    bufs = ((hx0, gs0), (hx1, gs1))

    def issue(jj, b):
        hxv, gs = bufs[b]
        pltpu.async_copy(hx.at[sv.at[jj]], hxv, gs)

    def work(jj, b):
        hxv, gs = bufs[b]
        pltpu.sync_copy(rad.at[pl.ds(base + jj * CH, CH)], rd0)
        pltpu.make_async_copy(hx.at[sv.at[jj]], hxv, gs).wait()

        @plsc.parallel_loop(0, CH, step=1, unroll=4)
        def mul(i):
            for k in range(D // 16):
                sl = pl.ds(k * 16, 16)
                hxv[i, sl] = hxv[i, sl] * rd0[i, sl]

        pltpu.sync_copy(hxv, aggS.at[dv.at[jj]], add=True)

    issue(0, 0)
    issue(1, 1)

    def pair(j2, carry):
        for b in range(2):
            jj = j2 * 2 + b
            work(jj, b)
            issue(jj + 2, b)
        return carry

    lax.fori_loop(0, NCHUNK // 2 - 1, pair, 0)
    work(NCHUNK - 2, 0)
    work(NCHUNK - 1, 1)

    plsc.subcore_barrier()
    pltpu.sync_copy(aggS.at[pl.ds(sid * ZROWS, ZROWS)],
                    agg2.at[cid, pl.ds(sid * ZROWS, ZROWS)])


def _sc_gather_mul_scatter(*args):
    return pl.kernel(
        _sc_gms_body,
        out_type=jax.ShapeDtypeStruct((NC, AGG_ROWS, D), jnp.float32),
        mesh=_sc_mesh(),
        scratch_types=[
            pltpu.VMEM((CH, D), jnp.float32),
            pltpu.VMEM((CH, D), jnp.float32),
            pltpu.VMEM((CH, D), jnp.float32),
            pltpu.VMEM((NCHUNK, CH), jnp.int32),
            pltpu.VMEM((NCHUNK, CH), jnp.int32),
            pltpu.VMEM_SHARED((AGG_ROWS, D), jnp.float32),
            pltpu.SemaphoreType.DMA,
            pltpu.SemaphoreType.DMA,
        ],
    )(*args)


_BE = 2048
_BN = 1000


def _rad_body(dv_ref, cen_ref, msk_ref, r1_ref, b1_ref, r2w_ref, b2_ref,
              r3_ref, b3_ref, out_ref):
    dv = dv_ref[...]
    r2 = jnp.sum(dv * dv, axis=1, keepdims=True)
    r = jnp.sqrt(r2 + 1e-12)
    width = NB / MAX_RADIUS
    basis = jnp.exp(-((r - cen_ref[...]) * width) ** 2)
    t = jnp.clip(r / MAX_RADIUS, 0.0, 1.0)
    cutoff = 0.5 * (jnp.cos(jnp.pi * t) + 1.0)
    basis = basis * cutoff * msk_ref[...]
    t1 = jax.nn.silu(
        jnp.dot(basis, r1_ref[...], preferred_element_type=jnp.float32)
        + b1_ref[...])
    t2 = jax.nn.silu(
        jnp.dot(t1, r2w_ref[...], preferred_element_type=jnp.float32)
        + b2_ref[...])
    out_ref[...] = (
        jnp.dot(t2, r3_ref[...], preferred_element_type=jnp.float32)
        + b3_ref[...])


def _tc_rad(dvec, centers_row, colmask_row, R1w, b1, R2w, b2, R3w, b3):
    grid = E_PAD // _BE
    return pl.pallas_call(
        _rad_body,
        grid=(grid,),
        in_specs=[
            pl.BlockSpec((_BE, DW), lambda g: (g, 0)),
            pl.BlockSpec((1, D), lambda g: (0, 0)),
            pl.BlockSpec((1, D), lambda g: (0, 0)),
            pl.BlockSpec((D, RN), lambda g: (0, 0)),
            pl.BlockSpec((1, RN), lambda g: (0, 0)),
            pl.BlockSpec((RN, RN), lambda g: (0, 0)),
            pl.BlockSpec((1, RN), lambda g: (0, 0)),
            pl.BlockSpec((RN, D), lambda g: (0, 0)),
            pl.BlockSpec((1, D), lambda g: (0, 0)),
        ],
        out_specs=pl.BlockSpec((_BE, D), lambda g: (g, 0)),
        out_shape=jax.ShapeDtypeStruct((E_PAD, D), jnp.float32),
    )(dvec, centers_row, colmask_row, R1w, b1, R2w, b2, R3w, b3)


def _hx0_body(x_ref, w_ref, out_ref):
    out_ref[...] = jnp.dot(x_ref[...], w_ref[...],
                           preferred_element_type=jnp.float32)


def _tc_hx0(x, W1l):
    return pl.pallas_call(
        _hx0_body,
        grid=(N // _BN,),
        in_specs=[
            pl.BlockSpec((_BN, D), lambda g: (g, 0)),
            pl.BlockSpec((D, D), lambda g: (0, 0)),
        ],
        out_specs=pl.BlockSpec((_BN, D), lambda g: (g, 0)),
        out_shape=jax.ShapeDtypeStruct((N, D), jnp.float32),
    )(x, W1l)


def _interp_hx0_body(oi_ref, of_ref, p_ref, w_ref, out_ref):
    pp = p_ref[0, 0]
    xts = (1.0 - pp) * oi_ref[...] + pp * of_ref[...]
    out_ref[...] = jnp.dot(xts, w_ref[...],
                           preferred_element_type=jnp.float32)


def _tc_interp_hx0(out_i, out_f, p2d, W1l):
    return pl.pallas_call(
        _interp_hx0_body,
        grid=(N // _BN,),
        in_specs=[
            pl.BlockSpec((_BN, D), lambda g: (g, 0)),
            pl.BlockSpec((_BN, D), lambda g: (g, 0)),
            pl.BlockSpec((1, 1), lambda g: (0, 0), memory_space=pltpu.SMEM),
            pl.BlockSpec((D, D), lambda g: (0, 0)),
        ],
        out_specs=pl.BlockSpec((_BN, D), lambda g: (g, 0)),
        out_shape=jax.ShapeDtypeStruct((N, D), jnp.float32),
    )(out_i, out_f, p2d, W1l)


def _mid_body(a0_ref, a1_ref, w2_ref, w1_ref, out_ref):
    agg = (a0_ref[0] + a1_ref[0]) * INV_SQRT_NEIGH
    h = jax.nn.silu(jnp.dot(agg, w2_ref[...],
                            preferred_element_type=jnp.float32))
    out_ref[...] = jnp.dot(h, w1_ref[...],
                           preferred_element_type=jnp.float32)


def _tc_mid(agg2, W2l, W1n):
    return pl.pallas_call(
        _mid_body,
        grid=(N // _BN,),
        in_specs=[
            pl.BlockSpec((1, _BN, D), lambda g: (0, g, 0)),
            pl.BlockSpec((1, _BN, D), lambda g: (1, g, 0)),
            pl.BlockSpec((D, D), lambda g: (0, 0)),
            pl.BlockSpec((D, D), lambda g: (0, 0)),
        ],
        out_specs=pl.BlockSpec((_BN, D), lambda g: (g, 0)),
        out_shape=jax.ShapeDtypeStruct((N, D), jnp.float32),
    )(agg2, agg2, W2l, W1n)


def _last_body(a0_ref, a1_ref, w2_ref, out_ref):
    agg = (a0_ref[0] + a1_ref[0]) * INV_SQRT_NEIGH
    out_ref[...] = jnp.dot(agg, w2_ref[...],
                           preferred_element_type=jnp.float32)


def _tc_last(agg2, W2l):
    return pl.pallas_call(
        _last_body,
        grid=(N // _BN,),
        in_specs=[
            pl.BlockSpec((1, _BN, D), lambda g: (0, g, 0)),
            pl.BlockSpec((1, _BN, D), lambda g: (1, g, 0)),
            pl.BlockSpec((D, D), lambda g: (0, 0)),
        ],
        out_specs=pl.BlockSpec((_BN, D), lambda g: (g, 0)),
        out_shape=jax.ShapeDtypeStruct((N, D), jnp.float32),
    )(agg2, agg2, W2l)


def _norm_body(x_ref, batch_ref, out_ref):
    x = x_ref[...]
    ss = jnp.sum(x * x, axis=1, keepdims=True)
    gid = lax.broadcasted_iota(jnp.int32, (N, NG), 1)
    seg = (batch_ref[...] == gid).astype(jnp.float32)
    ssg = jnp.sum(seg * ss, axis=0, keepdims=True)
    inv = 1.0 / jnp.sqrt(ssg + 1e-12)
    node_inv = jnp.sum(seg * inv, axis=1, keepdims=True)
    out_ref[...] = x * node_inv


def _tc_norm(x, batch2d):
    return pl.pallas_call(
        _norm_body,
        out_shape=jax.ShapeDtypeStruct((N, D), jnp.float32),
    )(x, batch2d)


def _prep_edges(edge_index):
    npad = E_PAD - E
    src = jnp.concatenate([edge_index[0], jnp.zeros((npad,), jnp.int32)])
    dst = jnp.concatenate([edge_index[1],
                           jnp.full((npad,), N, jnp.int32)])
    return src, dst


def _prep_pos(pos):
    return jnp.pad(pos, ((0, AGG_ROWS - N), (0, PW - 3)))


def _gate_net(pos, x0_hx, edge_index, net, W1, W2, R1w, b1, R2, b2, R3, b3,
              centers_row, colmask_row):
    src, dst = _prep_edges(edge_index)
    srcb = src.reshape(E_PAD // CH, CH)
    dstb = dst.reshape(E_PAD // CH, CH)
    posp = _prep_pos(pos)
    dvec = _sc_edge_dvec(posp, src.reshape(E_PAD // CHD, CHD),
                         dst.reshape(E_PAD // CHD, CHD))
    hx = x0_hx
    for l in range(L):
        rad = _tc_rad(dvec, centers_row, colmask_row, R1w[net, l],
                      b1[net, l].reshape(1, RN), R2[net, l],
                      b2[net, l].reshape(1, RN), R3[net, l],
                      b3[net, l].reshape(1, D))
        agg2 = _sc_gather_mul_scatter(hx, rad, srcb, dstb)
        if l < L - 1:
            hx = _tc_mid(agg2, W2[net, l], W1[net, l + 1])
        else:
            h = _tc_last(agg2, W2[net, l])
    return h


def kernel(pos, x, batch, edge_index, pos_final_state, x_final_state,
           edge_index_final_state, pos_interpolated_transition_state,
           edge_index_interpolated_transition_state, p, W1, W2, R1, b1, R2,
           b2, R3, b3):
    centers = jnp.linspace(0.0, MAX_RADIUS, NB)
    centers_row = jnp.pad(centers, (0, D - NB)).reshape(1, D)
    colmask_row = (jnp.arange(D) < NB).astype(jnp.float32).reshape(1, D)
    R1w = jnp.pad(R1, ((0, 0), (0, 0), (0, D - NB), (0, 0)))
    batch2d = batch.astype(jnp.int32).reshape(N, 1)
    p2d = p.reshape(1, 1)

    hx0_i = _tc_hx0(x, W1[0, 0])
    h_i = _gate_net(pos, hx0_i, edge_index, 0, W1, W2, R1w, b1, R2, b2, R3,
                    b3, centers_row, colmask_row)
    out_i = _tc_norm(h_i, batch2d)

    hx0_f = _tc_hx0(x_final_state, W1[1, 0])
    h_f = _gate_net(pos_final_state, hx0_f, edge_index_final_state, 1, W1,
                    W2, R1w, b1, R2, b2, R3, b3, centers_row, colmask_row)
    out_f = _tc_norm(h_f, batch2d)

    hx0_t = _tc_interp_hx0(out_i, out_f, p2d, W1[2, 0])
    h_t = _gate_net(pos_interpolated_transition_state, hx0_t,
                    edge_index_interpolated_transition_state, 2, W1, W2,
                    R1w, b1, R2, b2, R3, b3, centers_row, colmask_row)
    out_ts = _tc_norm(h_t, batch2d)
    return out_ts
